# baseline (speedup 1.0000x reference)
import functools

import jax
import jax.numpy as jnp
from jax import lax
from jax.experimental import pallas as pl
from jax.experimental.pallas import tpu as pltpu
from jax.experimental.pallas import tpu_sc as plsc

N = 10000
E = 320000
D = 128

NC = 2
NS = 16
NW = NC * NS
CHUNK = 128
CPT = 80
EPT = CHUNK * CPT
EP = EPT * NW
NPAD = 10240
RPT = NPAD // NS

def _sc_agg_body(with_deg, x_hbm, edges_hbm, *refs):
  if with_deg:
    (out_hbm, deg_hbm, ibuf0, ibuf1, buf0, buf1, ones_v, zvec_v, zbuf,
     acc_sh, deg_sh, semi0, semi1, semg0, semg1) = refs
  else:
    (out_hbm, ibuf0, ibuf1, buf0, buf1, ones_v, zvec_v, zbuf,
     acc_sh, deg_sh, semi0, semi1, semg0, semg1) = refs
    deg_hbm = None
  ibuf = (ibuf0, ibuf1)
  rbuf = (buf0, buf1)
  semi = (semi0, semi1)
  semg = (semg0, semg1)

  c = lax.axis_index("c")
  s = lax.axis_index("s")
  w = c * NS + s

  def start_idx(j, p):
    pltpu.async_copy(edges_hbm.at[w, j], ibuf[p], semi[p])

  def wait_idx(j, p):
    pltpu.make_async_copy(edges_hbm.at[w, j], ibuf[p], semi[p]).wait()

  def start_gather(p):
    pltpu.async_copy(x_hbm.at[ibuf[p].at[0]], rbuf[p], semg[p])

  def wait_gather(p):
    pltpu.make_async_copy(x_hbm.at[ibuf[p].at[0]], rbuf[p],
                          semg[p]).wait()

  def scatter(p):
    pltpu.sync_copy(rbuf[p], acc_sh.at[ibuf[p].at[1]], add=True)
    if with_deg:
      pltpu.sync_copy(ones_v, deg_sh.at[ibuf[p].at[1]], add=True)

  pltpu.sync_copy(edges_hbm.at[w, 0], ibuf0)
  start_idx(1, 1)
  start_gather(0)

  z16 = jnp.zeros((16,), jnp.float32)
  o16 = jnp.ones((16,), jnp.float32)
  for kk in range(CHUNK // 16):
    ones_v[pl.ds(kk * 16, 16)] = o16

  def zrow(r, _):
    for kk in range(D // 16):
      zbuf[r, pl.ds(kk * 16, 16)] = z16
    return 0
  lax.fori_loop(0, 64, zrow, 0)

  def zvecrow(r, _):
    zvec_v[pl.ds(r * 16, 16)] = z16
    return 0
  lax.fori_loop(0, RPT // 16, zvecrow, 0)

  base = s * RPT
  for k in range(RPT // 64):
    pltpu.sync_copy(zbuf, acc_sh.at[pl.ds(base + k * 64, 64)])
  pltpu.sync_copy(zvec_v, deg_sh.at[pl.ds(base, RPT)])

  plsc.subcore_barrier()

  def outer(it, _):
    j = it * 2
    for p in range(2):
      q = 1 - p
      wait_idx(j + p + 1, q)
      start_gather(q)
      wait_gather(p)
      scatter(p)
      start_idx(j + p + 2, p)
    return 0

  lax.fori_loop(0, (CPT - 2) // 2, outer, 0)

  wait_idx(CPT - 1, 1)
  start_gather(1)
  wait_gather(0)
  scatter(0)
  wait_gather(1)
  scatter(1)

  plsc.subcore_barrier()

  pltpu.sync_copy(acc_sh.at[pl.ds(base, RPT)], out_hbm.at[c, pl.ds(base, RPT)])
  if with_deg:
    pltpu.sync_copy(deg_sh.at[pl.ds(base, RPT)],
                    deg_hbm.at[pl.ds(c * NPAD + base, RPT)])


def _make_sc_agg(with_deg):
  out_type = [jax.ShapeDtypeStruct((NC, NPAD, D), jnp.float32)]
  if with_deg:
    out_type.append(jax.ShapeDtypeStruct((NC * NPAD,), jnp.float32))
  return pl.kernel(
      functools.partial(_sc_agg_body, with_deg),
      out_type=tuple(out_type) if with_deg else out_type[0],
      mesh=plsc.VectorSubcoreMesh(core_axis_name="c", subcore_axis_name="s"),
      scratch_types=(
          [pltpu.VMEM((2, CHUNK), jnp.int32)] * 2 +
          [pltpu.VMEM((CHUNK, D), jnp.float32)] * 2 +
          [pltpu.VMEM((CHUNK,), jnp.float32),
           pltpu.VMEM((RPT,), jnp.float32),
           pltpu.VMEM((64, D), jnp.float32),
           pltpu.VMEM_SHARED((NPAD, D), jnp.float32),
           pltpu.VMEM_SHARED((NPAD,), jnp.float32)] +
          [pltpu.SemaphoreType.DMA] * 4
      ),
      name="sage_sc_agg_deg" if with_deg else "sage_sc_agg",
  )


_sc_agg_deg = _make_sc_agg(True)
_sc_agg = _make_sc_agg(False)

BLK = 1000


def _tc_pre_body(x, wr, b, o):
  o[...] = jnp.dot(x[...], wr[...], preferred_element_type=jnp.float32,
                   precision=lax.Precision.HIGHEST) + b[...]


def _tc_pre(x, wr, b):
  return pl.pallas_call(
      _tc_pre_body,
      grid=(N // BLK,),
      in_specs=[
          pl.BlockSpec((BLK, D), lambda i: (i, 0)),
          pl.BlockSpec((D, D), lambda i: (0, 0)),
          pl.BlockSpec((1, D), lambda i: (0, 0)),
      ],
      out_specs=pl.BlockSpec((BLK, D), lambda i: (i, 0)),
      out_shape=jax.ShapeDtypeStruct((N, D), jnp.float32),
  )(x, wr, b)


def _tc1_body(pa, pb, dg, pre, wl, w2r, b2, oh, ohr):
  dtot = dg[:, 0:1] + dg[:, 1:2]
  rdeg = 1.0 / jnp.maximum(dtot, 1.0)
  mean = (pa[...] + pb[...]) * rdeg
  acc = jnp.dot(mean, wl[...], preferred_element_type=jnp.float32,
                precision=lax.Precision.HIGHEST)
  h = jnp.maximum(acc + pre[...], 0.0)
  oh[...] = h
  ohr[...] = jnp.dot(h, w2r[...], preferred_element_type=jnp.float32,
                     precision=lax.Precision.HIGHEST) + b2[...]


def _tc2_body(pa, pb, dg, pre, wl, o):
  dtot = dg[:, 0:1] + dg[:, 1:2]
  rdeg = 1.0 / jnp.maximum(dtot, 1.0)
  mean = (pa[...] + pb[...]) * rdeg
  z = jnp.dot(mean, wl[...], preferred_element_type=jnp.float32,
              precision=lax.Precision.HIGHEST) + pre[...]
  m = jnp.max(z, axis=1, keepdims=True)
  lse = jnp.log(jnp.sum(jnp.exp(z - m), axis=1, keepdims=True)) + m
  o[...] = z - lse


def _tc_layer1(pa, pb, dgt, pre, wl, w2r, b2):
  return pl.pallas_call(
      _tc1_body,
      grid=(N // BLK,),
      in_specs=[
          pl.BlockSpec((BLK, D), lambda i: (i, 0)),
          pl.BlockSpec((BLK, D), lambda i: (i, 0)),
          pl.BlockSpec((BLK, 2), lambda i: (i, 0)),
          pl.BlockSpec((BLK, D), lambda i: (i, 0)),
          pl.BlockSpec((D, D), lambda i: (0, 0)),
          pl.BlockSpec((D, D), lambda i: (0, 0)),
          pl.BlockSpec((1, D), lambda i: (0, 0)),
      ],
      out_specs=(pl.BlockSpec((BLK, D), lambda i: (i, 0)),
                 pl.BlockSpec((BLK, D), lambda i: (i, 0))),
      out_shape=(jax.ShapeDtypeStruct((N, D), jnp.float32),
                 jax.ShapeDtypeStruct((N, D), jnp.float32)),
  )(pa, pb, dgt, pre, wl, w2r, b2)


def _tc_layer2(pa, pb, dgt, pre, wl):
  return pl.pallas_call(
      _tc2_body,
      grid=(N // BLK,),
      in_specs=[
          pl.BlockSpec((BLK, D), lambda i: (i, 0)),
          pl.BlockSpec((BLK, D), lambda i: (i, 0)),
          pl.BlockSpec((BLK, 2), lambda i: (i, 0)),
          pl.BlockSpec((BLK, D), lambda i: (i, 0)),
          pl.BlockSpec((D, D), lambda i: (0, 0)),
      ],
      out_specs=pl.BlockSpec((BLK, D), lambda i: (i, 0)),
      out_shape=jax.ShapeDtypeStruct((N, D), jnp.float32),
  )(pa, pb, dgt, pre, wl)


def kernel(x, edge_index, W1_l, b1_l, W1_r, W2_l, b2_l, W2_r):
  src = edge_index[0]
  dst = edge_index[1]
  pad = EP - E
  ar = jnp.arange(pad, dtype=jnp.int32)
  pad_src = (ar * 37) % N
  pad_dst = N + ar % (NPAD - N)
  src_r = jnp.concatenate([src, pad_src]).reshape(NW, CPT, CHUNK)
  dst_r = jnp.concatenate([dst, pad_dst]).reshape(NW, CPT, CHUNK)
  edges_r = jnp.stack([src_r, dst_r], axis=2)

  xr = _tc_pre(x, W1_r, b1_l.reshape(1, D))
  p1, deg_flat = _sc_agg_deg(x, edges_r)
  dgt = jnp.transpose(deg_flat.reshape(NC, NPAD))
  h, hr = _tc_layer1(p1[0], p1[1], dgt, xr, W1_l, W2_r, b2_l.reshape(1, D))

  p2 = _sc_agg(h, edges_r)
  out = _tc_layer2(p2[0], p2[1], dgt, hr, W2_l)
  return out


# speedup vs baseline: 1.1068x; 1.1068x over previous
import functools

import jax
import jax.numpy as jnp
from jax import lax
from jax.experimental import pallas as pl
from jax.experimental.pallas import tpu as pltpu
from jax.experimental.pallas import tpu_sc as plsc

N = 10000
E = 320000
D = 128

NC = 2
NS = 16
NW = NC * NS
CHUNK = 128
CPT = 80
EPT = CHUNK * CPT
EP = EPT * NW
NPAD = 10240
RPT = NPAD // NS
NPAD2 = 10112
RPT2 = NPAD2 // NS

def _sc_agg_body(with_deg, x_hbm, edges_hbm, *refs):
  if with_deg:
    (out_hbm, deg_hbm, ibuf0, ibuf1, buf0, buf1, ones_v, zvec_v, zbuf,
     acc_sh, deg_sh, semi0, semi1, semg0, semg1) = refs
  else:
    (out_hbm, ibuf0, ibuf1, buf0, buf1, zbuf,
     acc_sh, semi0, semi1, semg0, semg1) = refs
    deg_hbm = ones_v = zvec_v = deg_sh = None
  ibuf = (ibuf0, ibuf1)
  rbuf = (buf0, buf1)
  semi = (semi0, semi1)
  semg = (semg0, semg1)

  c = lax.axis_index("c")
  s = lax.axis_index("s")
  w = c * NS + s

  def start_idx(j, p):
    pltpu.async_copy(edges_hbm.at[w, j], ibuf[p], semi[p])

  def wait_idx(j, p):
    pltpu.make_async_copy(edges_hbm.at[w, j], ibuf[p], semi[p]).wait()

  def start_gather(p):
    pltpu.async_copy(x_hbm.at[ibuf[p].at[0]], rbuf[p], semg[p])

  def wait_gather(p):
    pltpu.make_async_copy(x_hbm.at[ibuf[p].at[0]], rbuf[p],
                          semg[p]).wait()

  def scatter(p):
    pltpu.sync_copy(rbuf[p], acc_sh.at[ibuf[p].at[1]], add=True)
    if with_deg:
      pltpu.sync_copy(ones_v, deg_sh.at[ibuf[p].at[1]], add=True)

  pltpu.sync_copy(edges_hbm.at[w, 0], ibuf0)
  start_idx(1, 1)
  start_gather(0)

  z16 = jnp.zeros((16,), jnp.float32)
  if with_deg:
    o16 = jnp.ones((16,), jnp.float32)
    for kk in range(CHUNK // 16):
      ones_v[pl.ds(kk * 16, 16)] = o16

  def zrow(r, _):
    for kk in range(D // 16):
      zbuf[r, pl.ds(kk * 16, 16)] = z16
    return 0
  lax.fori_loop(0, 64, zrow, 0)

  if with_deg:
    def zvecrow(r, _):
      zvec_v[pl.ds(r * 16, 16)] = z16
      return 0
    lax.fori_loop(0, RPT // 16, zvecrow, 0)

  base = s * RPT
  for k in range(RPT // 64):
    pltpu.sync_copy(zbuf, acc_sh.at[pl.ds(base + k * 64, 64)])
  if with_deg:
    pltpu.sync_copy(zvec_v, deg_sh.at[pl.ds(base, RPT)])

  plsc.subcore_barrier()

  def outer(it, _):
    j = it * 2
    for p in range(2):
      q = 1 - p
      wait_idx(j + p + 1, q)
      start_gather(q)
      wait_gather(p)
      scatter(p)
      start_idx(j + p + 2, p)
    return 0

  lax.fori_loop(0, (CPT - 2) // 2, outer, 0)

  wait_idx(CPT - 1, 1)
  start_gather(1)
  wait_gather(0)
  scatter(0)
  wait_gather(1)
  scatter(1)

  plsc.subcore_barrier()

  pltpu.sync_copy(acc_sh.at[pl.ds(base, RPT)], out_hbm.at[c, pl.ds(base, RPT)])
  if with_deg:
    pltpu.sync_copy(deg_sh.at[pl.ds(base, RPT)],
                    deg_hbm.at[pl.ds(c * NPAD + base, RPT)])


def _sc_agg3_body(x_hbm, edges_hbm, out_hbm, ibuf0, ibuf1, ibuf2,
                  buf0, buf1, buf2, acc_sh,
                  semi0, semi1, semi2, semg0, semg1, semg2):
  ibuf = (ibuf0, ibuf1, ibuf2)
  rbuf = (buf0, buf1, buf2)
  semi = (semi0, semi1, semi2)
  semg = (semg0, semg1, semg2)

  c = lax.axis_index("c")
  s = lax.axis_index("s")
  w = c * NS + s

  def start_idx(j, b):
    pltpu.async_copy(edges_hbm.at[w, j], ibuf[b], semi[b])

  def wait_idx(j, b):
    pltpu.make_async_copy(edges_hbm.at[w, j], ibuf[b], semi[b]).wait()

  def start_gather(b):
    pltpu.async_copy(x_hbm.at[ibuf[b].at[0]], rbuf[b], semg[b])

  def wait_gather(b):
    pltpu.make_async_copy(x_hbm.at[ibuf[b].at[0]], rbuf[b],
                          semg[b]).wait()

  def scatter(b):
    pltpu.sync_copy(rbuf[b], acc_sh.at[ibuf[b].at[1]], add=True)

  pltpu.sync_copy(edges_hbm.at[w, 0], ibuf0)
  start_idx(1, 1)
  start_idx(2, 2)

  z16 = jnp.zeros((16,), jnp.float32)

  def zrow(r, _):
    for kk in range(D // 16):
      buf0[r, pl.ds(kk * 16, 16)] = z16
    return 0
  lax.fori_loop(0, CHUNK, zrow, 0)

  base = s * RPT2
  for k in range(4):
    pltpu.sync_copy(buf0, acc_sh.at[pl.ds(base + k * CHUNK, CHUNK)])
  pltpu.sync_copy(buf0.at[pl.ds(0, RPT2 - 4 * CHUNK)],
                  acc_sh.at[pl.ds(base + 4 * CHUNK, RPT2 - 4 * CHUNK)])

  start_gather(0)
  wait_idx(1, 1)
  start_gather(1)

  plsc.subcore_barrier()

  def outer(it, _):
    j0 = it * 3
    for b in range(3):
      j = j0 + b
      wait_gather(b)
      scatter(b)
      wait_idx(j + 2, (b + 2) % 3)
      start_gather((b + 2) % 3)
      @pl.when(j + 3 < CPT)
      def _():
        start_idx(j + 3, b)
    return 0

  lax.fori_loop(0, (CPT - 2) // 3, outer, 0)

  wait_gather(0)
  scatter(0)
  wait_gather(1)
  scatter(1)

  plsc.subcore_barrier()

  pltpu.sync_copy(acc_sh.at[pl.ds(base, RPT2)],
                  out_hbm.at[c, pl.ds(base, RPT2)])


_sc_agg3 = pl.kernel(
    _sc_agg3_body,
    out_type=jax.ShapeDtypeStruct((NC, NPAD2, D), jnp.float32),
    mesh=plsc.VectorSubcoreMesh(core_axis_name="c", subcore_axis_name="s"),
    scratch_types=(
        [pltpu.VMEM((2, CHUNK), jnp.int32)] * 3 +
        [pltpu.VMEM((CHUNK, D), jnp.float32)] * 3 +
        [pltpu.VMEM_SHARED((NPAD2, D), jnp.float32)] +
        [pltpu.SemaphoreType.DMA] * 6
    ),
    name="sage_sc_agg3",
)


def _make_sc_agg(with_deg):
  out_type = [jax.ShapeDtypeStruct((NC, NPAD, D), jnp.float32)]
  if with_deg:
    out_type.append(jax.ShapeDtypeStruct((NC * NPAD,), jnp.float32))
  return pl.kernel(
      functools.partial(_sc_agg_body, with_deg),
      out_type=tuple(out_type) if with_deg else out_type[0],
      mesh=plsc.VectorSubcoreMesh(core_axis_name="c", subcore_axis_name="s"),
      scratch_types=(
          [pltpu.VMEM((2, CHUNK), jnp.int32)] * 2 +
          [pltpu.VMEM((CHUNK, D), jnp.float32)] * 2 +
          ([pltpu.VMEM((CHUNK,), jnp.float32),
            pltpu.VMEM((RPT,), jnp.float32)]
           if with_deg else []) +
          [pltpu.VMEM((64, D), jnp.float32),
           pltpu.VMEM_SHARED((NPAD, D), jnp.float32)] +
          ([pltpu.VMEM_SHARED((NPAD,), jnp.float32)]
           if with_deg else []) +
          [pltpu.SemaphoreType.DMA] * 4
      ),
      name="sage_sc_agg_deg" if with_deg else "sage_sc_agg",
  )


_sc_agg_deg = _make_sc_agg(True)
_sc_agg = _make_sc_agg(False)

BLK = 1000


def _tc_pre_body(x, wr, b, o):
  o[...] = jnp.dot(x[...], wr[...], preferred_element_type=jnp.float32,
                   precision=lax.Precision.HIGHEST) + b[...]


def _tc_pre(x, wr, b):
  return pl.pallas_call(
      _tc_pre_body,
      grid=(N // BLK,),
      in_specs=[
          pl.BlockSpec((BLK, D), lambda i: (i, 0)),
          pl.BlockSpec((D, D), lambda i: (0, 0)),
          pl.BlockSpec((1, D), lambda i: (0, 0)),
      ],
      out_specs=pl.BlockSpec((BLK, D), lambda i: (i, 0)),
      out_shape=jax.ShapeDtypeStruct((N, D), jnp.float32),
  )(x, wr, b)


def _tc1_body(pa, pb, dg, pre, wl, o):
  dtot = dg[:, 0:1] + dg[:, 1:2]
  rdeg = 1.0 / jnp.maximum(dtot, 1.0)
  mean = (pa[...] + pb[...]) * rdeg
  acc = jnp.dot(mean, wl[...], preferred_element_type=jnp.float32,
                precision=lax.Precision.HIGHEST)
  o[...] = jnp.maximum(acc + pre[...], 0.0)


def _tc2_body(pa, pb, dg, pre, wl, o):
  dtot = dg[:, 0:1] + dg[:, 1:2]
  rdeg = 1.0 / jnp.maximum(dtot, 1.0)
  mean = (pa[...] + pb[...]) * rdeg
  z = jnp.dot(mean, wl[...], preferred_element_type=jnp.float32,
              precision=lax.Precision.HIGHEST) + pre[...]
  m = jnp.max(z, axis=1, keepdims=True)
  lse = jnp.log(jnp.sum(jnp.exp(z - m), axis=1, keepdims=True)) + m
  o[...] = z - lse


def _tc_layer(body, pa, pb, dgt, pre, wl):
  return pl.pallas_call(
      body,
      grid=(N // BLK,),
      in_specs=[
          pl.BlockSpec((BLK, D), lambda i: (i, 0)),
          pl.BlockSpec((BLK, D), lambda i: (i, 0)),
          pl.BlockSpec((BLK, 2), lambda i: (i, 0)),
          pl.BlockSpec((BLK, D), lambda i: (i, 0)),
          pl.BlockSpec((D, D), lambda i: (0, 0)),
      ],
      out_specs=pl.BlockSpec((BLK, D), lambda i: (i, 0)),
      out_shape=jax.ShapeDtypeStruct((N, D), jnp.float32),
  )(pa, pb, dgt, pre, wl)


def kernel(x, edge_index, W1_l, b1_l, W1_r, W2_l, b2_l, W2_r):
  src = edge_index[0]
  dst = edge_index[1]
  pad = EP - E
  ar = jnp.arange(pad, dtype=jnp.int32)
  pad_src = (ar * 37) % N
  pad_dst = N + ar % (NPAD2 - N)
  src_r = jnp.concatenate([src, pad_src]).reshape(NW, CPT, CHUNK)
  dst_r = jnp.concatenate([dst, pad_dst]).reshape(NW, CPT, CHUNK)
  edges_r = jnp.stack([src_r, dst_r], axis=2)

  xr = _tc_pre(x, W1_r, b1_l.reshape(1, D))
  p1, deg_flat = _sc_agg_deg(x, edges_r)
  dgt = jnp.transpose(deg_flat.reshape(NC, NPAD))
  h = _tc_layer(_tc1_body, p1[0], p1[1], dgt, xr, W1_l)

  hr = _tc_pre(h, W2_r, b2_l.reshape(1, D))
  p2 = _sc_agg3(h, edges_r)
  out = _tc_layer(_tc2_body, p2[0], p2[1], dgt, hr, W2_l)
  return out


# speedup vs baseline: 1.1170x; 1.0093x over previous
import functools

import jax
import jax.numpy as jnp
from jax import lax
from jax.experimental import pallas as pl
from jax.experimental.pallas import tpu as pltpu
from jax.experimental.pallas import tpu_sc as plsc

N = 10000
E = 320000
D = 128

NC = 2
NS = 16
NW = NC * NS
CHUNK = 128
CPT = 80
EPT = CHUNK * CPT
EP = EPT * NW
NPAD = 10240
RPT = NPAD // NS
NPAD2 = 10112
RPT2 = NPAD2 // NS
CHUNK1 = 120
CPT1 = 86
EPT1 = CHUNK1 * CPT1
EP1 = EPT1 * NW

def _sc_agg1_body(x_hbm, edges_hbm, out_hbm, deg_hbm,
                  ibuf0, ibuf1, ibuf2, buf0, buf1, buf2, ones_v, zvec_v,
                  acc_sh, deg_sh,
                  semi0, semi1, semi2, semg0, semg1, semg2):
  ibuf = (ibuf0, ibuf1, ibuf2)
  rbuf = (buf0, buf1, buf2)
  semi = (semi0, semi1, semi2)
  semg = (semg0, semg1, semg2)

  c = lax.axis_index("c")
  s = lax.axis_index("s")
  w = c * NS + s

  def start_idx(j, b):
    pltpu.async_copy(edges_hbm.at[w, j], ibuf[b], semi[b])

  def wait_idx(j, b):
    pltpu.make_async_copy(edges_hbm.at[w, j], ibuf[b], semi[b]).wait()

  def start_gather(b):
    pltpu.async_copy(x_hbm.at[ibuf[b].at[0]], rbuf[b], semg[b])

  def wait_gather(b):
    pltpu.make_async_copy(x_hbm.at[ibuf[b].at[0]], rbuf[b],
                          semg[b]).wait()

  def scatter(b):
    pltpu.sync_copy(rbuf[b], acc_sh.at[ibuf[b].at[1]], add=True)
    pltpu.sync_copy(ones_v.at[pl.ds(0, CHUNK1)], deg_sh.at[ibuf[b].at[1]],
                    add=True)

  pltpu.sync_copy(edges_hbm.at[w, 0], ibuf0)
  start_idx(1, 1)
  start_idx(2, 2)

  z16 = jnp.zeros((16,), jnp.float32)
  o16 = jnp.ones((16,), jnp.float32)
  for kk in range(8):
    ones_v[pl.ds(kk * 16, 16)] = o16
    zvec_v[pl.ds(kk * 16, 16)] = z16

  def zrow(r, _):
    for kk in range(D // 16):
      buf0[r, pl.ds(kk * 16, 16)] = z16
    return 0
  lax.fori_loop(0, CHUNK1, zrow, 0)

  base = s * RPT2
  for k in range(5):
    pltpu.sync_copy(buf0, acc_sh.at[pl.ds(base + k * CHUNK1, CHUNK1)])
  pltpu.sync_copy(buf0.at[pl.ds(0, RPT2 - 5 * CHUNK1)],
                  acc_sh.at[pl.ds(base + 5 * CHUNK1, RPT2 - 5 * CHUNK1)])
  dbase = s * RPT
  for k in range(RPT // 128):
    pltpu.sync_copy(zvec_v, deg_sh.at[pl.ds(dbase + k * 128, 128)])

  start_gather(0)
  wait_idx(1, 1)
  start_gather(1)

  plsc.subcore_barrier()

  def outer(it, _):
    j0 = it * 3
    for b in range(3):
      j = j0 + b
      wait_gather(b)
      scatter(b)
      wait_idx(j + 2, (b + 2) % 3)
      start_gather((b + 2) % 3)
      @pl.when(j + 3 < CPT1)
      def _():
        start_idx(j + 3, b)
    return 0

  lax.fori_loop(0, (CPT1 - 2) // 3, outer, 0)

  wait_gather(0)
  scatter(0)
  wait_gather(1)
  scatter(1)

  plsc.subcore_barrier()

  pltpu.sync_copy(acc_sh.at[pl.ds(base, RPT2)],
                  out_hbm.at[c, pl.ds(base, RPT2)])
  pltpu.sync_copy(deg_sh.at[pl.ds(dbase, RPT)],
                  deg_hbm.at[pl.ds(c * NPAD + dbase, RPT)])


_sc_agg1 = pl.kernel(
    _sc_agg1_body,
    out_type=(jax.ShapeDtypeStruct((NC, NPAD2, D), jnp.float32),
              jax.ShapeDtypeStruct((NC * NPAD,), jnp.float32)),
    mesh=plsc.VectorSubcoreMesh(core_axis_name="c", subcore_axis_name="s"),
    scratch_types=(
        [pltpu.VMEM((2, CHUNK1), jnp.int32)] * 3 +
        [pltpu.VMEM((CHUNK1, D), jnp.float32)] * 3 +
        [pltpu.VMEM((128,), jnp.float32),
         pltpu.VMEM((128,), jnp.float32),
         pltpu.VMEM_SHARED((NPAD2, D), jnp.float32),
         pltpu.VMEM_SHARED((NPAD,), jnp.float32)] +
        [pltpu.SemaphoreType.DMA] * 6
    ),
    name="sage_sc_agg1",
)


def _sc_agg3_body(x_hbm, edges_hbm, out_hbm, ibuf0, ibuf1, ibuf2,
                  buf0, buf1, buf2, acc_sh,
                  semi0, semi1, semi2, semg0, semg1, semg2):
  ibuf = (ibuf0, ibuf1, ibuf2)
  rbuf = (buf0, buf1, buf2)
  semi = (semi0, semi1, semi2)
  semg = (semg0, semg1, semg2)

  c = lax.axis_index("c")
  s = lax.axis_index("s")
  w = c * NS + s

  def start_idx(j, b):
    pltpu.async_copy(edges_hbm.at[w, j], ibuf[b], semi[b])

  def wait_idx(j, b):
    pltpu.make_async_copy(edges_hbm.at[w, j], ibuf[b], semi[b]).wait()

  def start_gather(b):
    pltpu.async_copy(x_hbm.at[ibuf[b].at[0]], rbuf[b], semg[b])

  def wait_gather(b):
    pltpu.make_async_copy(x_hbm.at[ibuf[b].at[0]], rbuf[b],
                          semg[b]).wait()

  def scatter(b):
    pltpu.sync_copy(rbuf[b], acc_sh.at[ibuf[b].at[1]], add=True)

  pltpu.sync_copy(edges_hbm.at[w, 0], ibuf0)
  start_idx(1, 1)
  start_idx(2, 2)

  z16 = jnp.zeros((16,), jnp.float32)

  def zrow(r, _):
    for kk in range(D // 16):
      buf0[r, pl.ds(kk * 16, 16)] = z16
    return 0
  lax.fori_loop(0, CHUNK, zrow, 0)

  base = s * RPT2
  for k in range(4):
    pltpu.sync_copy(buf0, acc_sh.at[pl.ds(base + k * CHUNK, CHUNK)])
  pltpu.sync_copy(buf0.at[pl.ds(0, RPT2 - 4 * CHUNK)],
                  acc_sh.at[pl.ds(base + 4 * CHUNK, RPT2 - 4 * CHUNK)])

  start_gather(0)
  wait_idx(1, 1)
  start_gather(1)

  plsc.subcore_barrier()

  def outer(it, _):
    j0 = it * 3
    for b in range(3):
      j = j0 + b
      wait_gather(b)
      scatter(b)
      wait_idx(j + 2, (b + 2) % 3)
      start_gather((b + 2) % 3)
      @pl.when(j + 3 < CPT)
      def _():
        start_idx(j + 3, b)
    return 0

  lax.fori_loop(0, (CPT - 2) // 3, outer, 0)

  wait_gather(0)
  scatter(0)
  wait_gather(1)
  scatter(1)

  plsc.subcore_barrier()

  pltpu.sync_copy(acc_sh.at[pl.ds(base, RPT2)],
                  out_hbm.at[c, pl.ds(base, RPT2)])


_sc_agg3 = pl.kernel(
    _sc_agg3_body,
    out_type=jax.ShapeDtypeStruct((NC, NPAD2, D), jnp.float32),
    mesh=plsc.VectorSubcoreMesh(core_axis_name="c", subcore_axis_name="s"),
    scratch_types=(
        [pltpu.VMEM((2, CHUNK), jnp.int32)] * 3 +
        [pltpu.VMEM((CHUNK, D), jnp.float32)] * 3 +
        [pltpu.VMEM_SHARED((NPAD2, D), jnp.float32)] +
        [pltpu.SemaphoreType.DMA] * 6
    ),
    name="sage_sc_agg3",
)


BLK = 1000


def _tc_pre_body(x, wr, b, o):
  o[...] = jnp.dot(x[...], wr[...], preferred_element_type=jnp.float32,
                   precision=lax.Precision.HIGHEST) + b[...]


def _tc_pre(x, wr, b):
  return pl.pallas_call(
      _tc_pre_body,
      grid=(N // BLK,),
      in_specs=[
          pl.BlockSpec((BLK, D), lambda i: (i, 0)),
          pl.BlockSpec((D, D), lambda i: (0, 0)),
          pl.BlockSpec((1, D), lambda i: (0, 0)),
      ],
      out_specs=pl.BlockSpec((BLK, D), lambda i: (i, 0)),
      out_shape=jax.ShapeDtypeStruct((N, D), jnp.float32),
  )(x, wr, b)


def _tc1_body(pa, pb, dg, pre, wl, o):
  dtot = dg[:, 0:1] + dg[:, 1:2]
  rdeg = 1.0 / jnp.maximum(dtot, 1.0)
  mean = (pa[...] + pb[...]) * rdeg
  acc = jnp.dot(mean, wl[...], preferred_element_type=jnp.float32,
                precision=lax.Precision.HIGHEST)
  o[...] = jnp.maximum(acc + pre[...], 0.0)


def _tc2_body(pa, pb, dg, pre, wl, o):
  dtot = dg[:, 0:1] + dg[:, 1:2]
  rdeg = 1.0 / jnp.maximum(dtot, 1.0)
  mean = (pa[...] + pb[...]) * rdeg
  z = jnp.dot(mean, wl[...], preferred_element_type=jnp.float32,
              precision=lax.Precision.HIGHEST) + pre[...]
  m = jnp.max(z, axis=1, keepdims=True)
  lse = jnp.log(jnp.sum(jnp.exp(z - m), axis=1, keepdims=True)) + m
  o[...] = z - lse


def _tc_layer(body, pa, pb, dgt, pre, wl):
  return pl.pallas_call(
      body,
      grid=(N // BLK,),
      in_specs=[
          pl.BlockSpec((BLK, D), lambda i: (i, 0)),
          pl.BlockSpec((BLK, D), lambda i: (i, 0)),
          pl.BlockSpec((BLK, 2), lambda i: (i, 0)),
          pl.BlockSpec((BLK, D), lambda i: (i, 0)),
          pl.BlockSpec((D, D), lambda i: (0, 0)),
      ],
      out_specs=pl.BlockSpec((BLK, D), lambda i: (i, 0)),
      out_shape=jax.ShapeDtypeStruct((N, D), jnp.float32),
  )(pa, pb, dgt, pre, wl)


def kernel(x, edge_index, W1_l, b1_l, W1_r, W2_l, b2_l, W2_r):
  src = edge_index[0]
  dst = edge_index[1]

  def padded_edges(ept, cpt, chunk):
    pad = ept * NW - E
    ar = jnp.arange(pad, dtype=jnp.int32)
    pad_src = (ar * 37) % N
    pad_dst = N + ar % (NPAD2 - N)
    src_r = jnp.concatenate([src, pad_src]).reshape(NW, cpt, chunk)
    dst_r = jnp.concatenate([dst, pad_dst]).reshape(NW, cpt, chunk)
    return jnp.stack([src_r, dst_r], axis=2)

  edges1_r = padded_edges(EPT1, CPT1, CHUNK1)
  edges_r = padded_edges(EPT, CPT, CHUNK)

  xr = _tc_pre(x, W1_r, b1_l.reshape(1, D))
  p1, deg_flat = _sc_agg1(x, edges1_r)
  dgt = jnp.transpose(deg_flat.reshape(NC, NPAD))
  h = _tc_layer(_tc1_body, p1[0], p1[1], dgt, xr, W1_l)

  hr = _tc_pre(h, W2_r, b2_l.reshape(1, D))
  p2 = _sc_agg3(h, edges_r)
  out = _tc_layer(_tc2_body, p2[0], p2[1], dgt, hr, W2_l)
  return out


# speedup vs baseline: 1.1243x; 1.0065x over previous
import functools

import jax
import jax.numpy as jnp
from jax import lax
from jax.experimental import pallas as pl
from jax.experimental.pallas import tpu as pltpu
from jax.experimental.pallas import tpu_sc as plsc

N = 10000
E = 320000
D = 128

NC = 2
NS = 16
NW = NC * NS
CHUNK = 128
CPT = 80
EPT = CHUNK * CPT
EP = EPT * NW
NPAD = 10240
RPT = NPAD // NS
NPAD2 = 10112
RPT2 = NPAD2 // NS
CHUNK1 = 120
CPT1 = 86
EPT1 = CHUNK1 * CPT1
EP1 = EPT1 * NW

def _sc_agg1_body(x_hbm, edges_hbm, out_hbm, deg_hbm,
                  ibuf0, ibuf1, ibuf2, buf0, buf1, buf2, ones_v, zvec_v,
                  acc_sh, deg_sh,
                  semi0, semi1, semi2, semg0, semg1, semg2):
  ibuf = (ibuf0, ibuf1, ibuf2)
  rbuf = (buf0, buf1, buf2)
  semi = (semi0, semi1, semi2)
  semg = (semg0, semg1, semg2)

  c = lax.axis_index("c")
  s = lax.axis_index("s")
  w = c * NS + s

  def start_idx(j, b):
    pltpu.async_copy(edges_hbm.at[w, j], ibuf[b], semi[b])

  def wait_idx(j, b):
    pltpu.make_async_copy(edges_hbm.at[w, j], ibuf[b], semi[b]).wait()

  def start_gather(b):
    pltpu.async_copy(x_hbm.at[ibuf[b].at[0]], rbuf[b], semg[b])

  def wait_gather(b):
    pltpu.make_async_copy(x_hbm.at[ibuf[b].at[0]], rbuf[b],
                          semg[b]).wait()

  def scatter(b):
    pltpu.sync_copy(rbuf[b], acc_sh.at[ibuf[b].at[1]], add=True)
    pltpu.sync_copy(ones_v.at[pl.ds(0, CHUNK1)], deg_sh.at[ibuf[b].at[1]],
                    add=True)

  pltpu.sync_copy(edges_hbm.at[w, 0], ibuf0)
  start_idx(1, 1)
  start_idx(2, 2)

  z16 = jnp.zeros((16,), jnp.float32)
  o16 = jnp.ones((16,), jnp.float32)
  for kk in range(8):
    ones_v[pl.ds(kk * 16, 16)] = o16
    zvec_v[pl.ds(kk * 16, 16)] = z16

  def zrow(r, _):
    for kk in range(D // 16):
      buf0[r, pl.ds(kk * 16, 16)] = z16
    return 0
  lax.fori_loop(0, CHUNK1, zrow, 0)

  base = s * RPT2
  for k in range(5):
    pltpu.sync_copy(buf0, acc_sh.at[pl.ds(base + k * CHUNK1, CHUNK1)])
  pltpu.sync_copy(buf0.at[pl.ds(0, RPT2 - 5 * CHUNK1)],
                  acc_sh.at[pl.ds(base + 5 * CHUNK1, RPT2 - 5 * CHUNK1)])
  dbase = s * RPT
  for k in range(RPT // 128):
    pltpu.sync_copy(zvec_v, deg_sh.at[pl.ds(dbase + k * 128, 128)])

  start_gather(0)
  wait_idx(1, 1)
  start_gather(1)

  plsc.subcore_barrier()

  def outer(it, _):
    j0 = it * 3
    for b in range(3):
      j = j0 + b
      wait_gather(b)
      scatter(b)
      wait_idx(j + 2, (b + 2) % 3)
      start_gather((b + 2) % 3)
      @pl.when(j + 3 < CPT1)
      def _():
        start_idx(j + 3, b)
    return 0

  lax.fori_loop(0, (CPT1 - 2) // 3, outer, 0)

  wait_gather(0)
  scatter(0)
  wait_gather(1)
  scatter(1)

  plsc.subcore_barrier()

  pltpu.sync_copy(acc_sh.at[pl.ds(base, RPT2)],
                  out_hbm.at[c, pl.ds(base, RPT2)])
  pltpu.sync_copy(deg_sh.at[pl.ds(dbase, RPT)],
                  deg_hbm.at[pl.ds(c * NPAD + dbase, RPT)])


_sc_agg1 = pl.kernel(
    _sc_agg1_body,
    out_type=(jax.ShapeDtypeStruct((NC, NPAD2, D), jnp.float32),
              jax.ShapeDtypeStruct((NC * NPAD,), jnp.float32)),
    mesh=plsc.VectorSubcoreMesh(core_axis_name="c", subcore_axis_name="s"),
    scratch_types=(
        [pltpu.VMEM((2, CHUNK1), jnp.int32)] * 3 +
        [pltpu.VMEM((CHUNK1, D), jnp.float32)] * 3 +
        [pltpu.VMEM((128,), jnp.float32),
         pltpu.VMEM((128,), jnp.float32),
         pltpu.VMEM_SHARED((NPAD2, D), jnp.float32),
         pltpu.VMEM_SHARED((NPAD,), jnp.float32)] +
        [pltpu.SemaphoreType.DMA] * 6
    ),
    name="sage_sc_agg1",
)


def _sc_agg3_body(x_hbm, edges_hbm, out_hbm, ibuf0, ibuf1, ibuf2,
                  buf0, buf1, buf2, acc_sh,
                  semi0, semi1, semi2, semg0, semg1, semg2):
  ibuf = (ibuf0, ibuf1, ibuf2)
  rbuf = (buf0, buf1, buf2)
  semi = (semi0, semi1, semi2)
  semg = (semg0, semg1, semg2)

  c = lax.axis_index("c")
  s = lax.axis_index("s")
  w = c * NS + s

  def start_idx(j, b):
    pltpu.async_copy(edges_hbm.at[w, j], ibuf[b], semi[b])

  def wait_idx(j, b):
    pltpu.make_async_copy(edges_hbm.at[w, j], ibuf[b], semi[b]).wait()

  def start_gather(b):
    pltpu.async_copy(x_hbm.at[ibuf[b].at[0]], rbuf[b], semg[b])

  def wait_gather(b):
    pltpu.make_async_copy(x_hbm.at[ibuf[b].at[0]], rbuf[b],
                          semg[b]).wait()

  def scatter(b):
    pltpu.sync_copy(rbuf[b], acc_sh.at[ibuf[b].at[1]], add=True)

  pltpu.sync_copy(edges_hbm.at[w, 0], ibuf0)
  start_idx(1, 1)
  start_idx(2, 2)

  z16 = jnp.zeros((16,), jnp.float32)

  def zrow(r, _):
    for kk in range(D // 16):
      buf0[r, pl.ds(kk * 16, 16)] = z16
    return 0
  lax.fori_loop(0, CHUNK1, zrow, 0)

  base = s * RPT2
  for k in range(5):
    pltpu.sync_copy(buf0, acc_sh.at[pl.ds(base + k * CHUNK1, CHUNK1)])
  pltpu.sync_copy(buf0.at[pl.ds(0, RPT2 - 5 * CHUNK1)],
                  acc_sh.at[pl.ds(base + 5 * CHUNK1, RPT2 - 5 * CHUNK1)])

  start_gather(0)
  wait_idx(1, 1)
  start_gather(1)

  plsc.subcore_barrier()

  def outer(it, _):
    j0 = it * 3
    for b in range(3):
      j = j0 + b
      wait_gather(b)
      scatter(b)
      wait_idx(j + 2, (b + 2) % 3)
      start_gather((b + 2) % 3)
      @pl.when(j + 3 < CPT1)
      def _():
        start_idx(j + 3, b)
    return 0

  lax.fori_loop(0, (CPT1 - 2) // 3, outer, 0)

  wait_gather(0)
  scatter(0)
  wait_gather(1)
  scatter(1)

  plsc.subcore_barrier()

  pltpu.sync_copy(acc_sh.at[pl.ds(base, RPT2)],
                  out_hbm.at[c, pl.ds(base, RPT2)])


_sc_agg3 = pl.kernel(
    _sc_agg3_body,
    out_type=jax.ShapeDtypeStruct((NC, NPAD2, D), jnp.float32),
    mesh=plsc.VectorSubcoreMesh(core_axis_name="c", subcore_axis_name="s"),
    scratch_types=(
        [pltpu.VMEM((2, CHUNK1), jnp.int32)] * 3 +
        [pltpu.VMEM((CHUNK1, D), jnp.float32)] * 3 +
        [pltpu.VMEM_SHARED((NPAD2, D), jnp.float32)] +
        [pltpu.SemaphoreType.DMA] * 6
    ),
    name="sage_sc_agg3",
)


BLK = 1000


def _tc_pre_body(x, wr, b, o):
  o[...] = jnp.dot(x[...], wr[...], preferred_element_type=jnp.float32,
                   precision=lax.Precision.HIGHEST) + b[...]


def _tc_pre(x, wr, b):
  return pl.pallas_call(
      _tc_pre_body,
      grid=(N // BLK,),
      in_specs=[
          pl.BlockSpec((BLK, D), lambda i: (i, 0)),
          pl.BlockSpec((D, D), lambda i: (0, 0)),
          pl.BlockSpec((1, D), lambda i: (0, 0)),
      ],
      out_specs=pl.BlockSpec((BLK, D), lambda i: (i, 0)),
      out_shape=jax.ShapeDtypeStruct((N, D), jnp.float32),
  )(x, wr, b)


def _tc1_body(pa, pb, dg, pre, wl, o):
  dtot = dg[:, 0:1] + dg[:, 1:2]
  rdeg = 1.0 / jnp.maximum(dtot, 1.0)
  mean = (pa[...] + pb[...]) * rdeg
  acc = jnp.dot(mean, wl[...], preferred_element_type=jnp.float32,
                precision=lax.Precision.HIGHEST)
  o[...] = jnp.maximum(acc + pre[...], 0.0)


def _tc2_body(pa, pb, dg, pre, wl, o):
  dtot = dg[:, 0:1] + dg[:, 1:2]
  rdeg = 1.0 / jnp.maximum(dtot, 1.0)
  mean = (pa[...] + pb[...]) * rdeg
  z = jnp.dot(mean, wl[...], preferred_element_type=jnp.float32,
              precision=lax.Precision.HIGHEST) + pre[...]
  m = jnp.max(z, axis=1, keepdims=True)
  lse = jnp.log(jnp.sum(jnp.exp(z - m), axis=1, keepdims=True)) + m
  o[...] = z - lse


def _tc_layer(body, pa, pb, dgt, pre, wl):
  return pl.pallas_call(
      body,
      grid=(N // BLK,),
      in_specs=[
          pl.BlockSpec((BLK, D), lambda i: (i, 0)),
          pl.BlockSpec((BLK, D), lambda i: (i, 0)),
          pl.BlockSpec((BLK, 2), lambda i: (i, 0)),
          pl.BlockSpec((BLK, D), lambda i: (i, 0)),
          pl.BlockSpec((D, D), lambda i: (0, 0)),
      ],
      out_specs=pl.BlockSpec((BLK, D), lambda i: (i, 0)),
      out_shape=jax.ShapeDtypeStruct((N, D), jnp.float32),
  )(pa, pb, dgt, pre, wl)


def kernel(x, edge_index, W1_l, b1_l, W1_r, W2_l, b2_l, W2_r):
  src = edge_index[0]
  dst = edge_index[1]

  def padded_edges(ept, cpt, chunk):
    pad = ept * NW - E
    ar = jnp.arange(pad, dtype=jnp.int32)
    pad_src = (ar * 37) % N
    pad_dst = N + ar % (NPAD2 - N)
    src_r = jnp.concatenate([src, pad_src]).reshape(NW, cpt, chunk)
    dst_r = jnp.concatenate([dst, pad_dst]).reshape(NW, cpt, chunk)
    return jnp.stack([src_r, dst_r], axis=2)

  edges1_r = padded_edges(EPT1, CPT1, CHUNK1)

  xr = _tc_pre(x, W1_r, b1_l.reshape(1, D))
  p1, deg_flat = _sc_agg1(x, edges1_r)
  dgt = jnp.transpose(deg_flat.reshape(NC, NPAD))
  h = _tc_layer(_tc1_body, p1[0], p1[1], dgt, xr, W1_l)

  hr = _tc_pre(h, W2_r, b2_l.reshape(1, D))
  p2 = _sc_agg3(h, edges1_r)
  out = _tc_layer(_tc2_body, p2[0], p2[1], dgt, hr, W2_l)
  return out
